# Initial kernel scaffold; baseline (speedup 1.0000x reference)
import jax
import jax.numpy as jnp
from jax.experimental import pallas as pl


def kernel(points1, points2, lengths1, lengths2):
    raise NotImplementedError("write your pallas kernel here")


# speedup vs baseline: 5.0389x; 5.0389x over previous
import functools

import jax
import jax.numpy as jnp
import numpy as np
from jax import lax
from jax.experimental import pallas as pl
from jax.experimental.pallas import tpu as pltpu
from jax.experimental.pallas import tpu_sc as plsc

K = 32
N1 = 10000
N2 = 10000
P = 10240
SHARD = 640
NCHUNK = SHARD // 16
NCELL = 1008
QW = P // 32
SENT = 1 << 30
R2 = np.float32(0.01)
MARGIN = np.float32(0.100002)


def _cell1d(v):
    return jnp.clip((v * jnp.float32(10.0)).astype(jnp.int32), 0, 9)


def _sc_ball_query(qx, qy, qz, px, py, pz):
    mesh = plsc.VectorSubcoreMesh(core_axis_name="c", subcore_axis_name="s")
    f32, i32 = jnp.float32, jnp.int32

    @functools.partial(
        pl.kernel,
        mesh=mesh,
        compiler_params=pltpu.CompilerParams(needs_layout_passes=False),
        out_type=[
            jax.ShapeDtypeStruct((P, K), i32),
            jax.ShapeDtypeStruct((P,), i32),
            jax.ShapeDtypeStruct((P, 3 * K), f32),
        ],
        scratch_types=[
            pltpu.VMEM((P,), f32),
            pltpu.VMEM((P,), f32),
            pltpu.VMEM((P,), f32),
            pltpu.VMEM((P,), f32),
            pltpu.VMEM((P,), f32),
            pltpu.VMEM((P,), f32),
            pltpu.VMEM((P,), i32),
            pltpu.VMEM((P,), i32),
            pltpu.VMEM((16, NCELL), i32),
            pltpu.VMEM((NCELL,), i32),
            pltpu.VMEM((NCELL,), i32),
            pltpu.VMEM((SHARD,), i32),
            pltpu.VMEM((SHARD,), i32),
            pltpu.VMEM((SHARD,), i32),
            pltpu.VMEM((SHARD // 128, 128), i32),
            pltpu.VMEM((SHARD,), i32),
            pltpu.VMEM((QW,), f32),
            pltpu.VMEM((QW,), f32),
            pltpu.VMEM((QW,), f32),
            pltpu.VMEM((QW,), i32),
            pltpu.VMEM((K,), i32),
            pltpu.VMEM((3 * K,), f32),
            pltpu.VMEM_SHARED((P + 4096,), f32),
            pltpu.VMEM_SHARED((P + 4096,), f32),
            pltpu.VMEM_SHARED((P + 4096,), f32),
            pltpu.VMEM_SHARED((P + 4096,), i32),
            pltpu.VMEM_SHARED((24, NCELL), i32),
            pltpu.VMEM_SHARED((16384,), i32),
        ],
    )
    def kfn(qx_h, qy_h, qz_h, px_h, py_h, pz_h,
            map_o, num_o, out_o,
            ox, oy, oz, sx, sy, sz, sidxb, surv, hist16, offs, cur,
            cellb, priorb, totb, posb, idxsh,
            qxb, qyb, qzb, numb, mrow, orow,
            ssx, ssy, ssz, ssidx, shist, spad):
        c = lax.axis_index("c")
        s = lax.axis_index("s")
        wid = c * 16 + s
        IOTA = lax.iota(i32, 16)
        Z16 = jnp.zeros((16,), i32)

        def zero_body(t, carry):
            plsc.store_scatter(cur, [t * 16 + IOTA], Z16)
            return carry

        lax.fori_loop(0, NCELL // 16, zero_body, 0)

        pltpu.sync_copy(px_h, ox)
        pltpu.sync_copy(py_h, oy)
        pltpu.sync_copy(pz_h, oz)
        pltpu.sync_copy(qx_h.at[pl.ds(wid * QW, QW)], qxb)
        pltpu.sync_copy(qy_h.at[pl.ds(wid * QW, QW)], qyb)
        pltpu.sync_copy(qz_h.at[pl.ds(wid * QW, QW)], qzb)

        def hist_body(j, carry):
            loc = j * 16 + IOTA
            gv = s * SHARD + loc
            m = gv < N2
            xv = plsc.load_gather(ox, [gv])
            yv = plsc.load_gather(oy, [gv])
            zv = plsc.load_gather(oz, [gv])
            cellv = _cell1d(xv) * 100 + _cell1d(yv) * 10 + _cell1d(zv)
            key = jnp.where(m, cellv, -1 - IOTA)
            prior = Z16
            after = Z16
            for k in range(1, 16):
                fwd = jnp.take_along_axis(
                    key, jnp.clip(IOTA - k, 0, 15), axis=0,
                    mode="promise_in_bounds")
                bwd = jnp.take_along_axis(
                    key, jnp.clip(IOTA + k, 0, 15), axis=0,
                    mode="promise_in_bounds")
                prior = prior + jnp.where((fwd == key) & (IOTA >= k), 1, 0)
                after = after + jnp.where((bwd == key) & (IOTA <= 15 - k), 1, 0)
            tot = prior + after + 1
            cellsafe = jnp.where(m, cellv, 1000 + IOTA)
            hv = plsc.load_gather(cur, [cellsafe])
            plsc.store_scatter(cur, [cellsafe], hv + tot, mask=m)
            plsc.store_scatter(cellb, [loc], cellsafe)
            plsc.store_scatter(priorb, [loc], prior)
            plsc.store_scatter(totb, [loc], tot)
            plsc.store_scatter(idxsh, [loc], gv)
            return carry

        lax.fori_loop(0, NCHUNK, hist_body, 0)

        pltpu.sync_copy(cur, shist.at[s])
        plsc.subcore_barrier()
        pltpu.sync_copy(shist.at[pl.ds(0, 16)], hist16)

        def offs_body(t, run):
            bidx = t * 16 + IOTA
            hsum = Z16
            prior_s = Z16
            for s2 in range(16):
                h = plsc.load_gather(hist16, [Z16 + s2, bidx])
                hsum = hsum + h
                prior_s = prior_s + jnp.where(s2 < s, h, 0)
            incl = plsc.cumsum(hsum)
            excl = incl - hsum
            plsc.store_scatter(offs, [bidx], run + excl)
            plsc.store_scatter(cur, [bidx], run + excl + prior_s)
            return run + jnp.max(incl)

        lax.fori_loop(0, NCELL // 16, offs_body, jnp.int32(0))

        def pos_body(j, carry):
            loc = j * 16 + IOTA
            gv = s * SHARD + loc
            m = gv < N2
            cellsafe = plsc.load_gather(cellb, [loc])
            prior = plsc.load_gather(priorb, [loc])
            tot = plsc.load_gather(totb, [loc])
            c0 = plsc.load_gather(cur, [cellsafe])
            pos = jnp.where(m, c0 + prior, P - 16 + IOTA)
            plsc.store_scatter(posb, [loc >> 7, loc & 127], pos)
            plsc.store_scatter(cur, [cellsafe], c0 + tot, mask=m)
            return carry

        lax.fori_loop(0, NCHUNK, pos_body, 0)

        base = s * SHARD
        for t in range(SHARD // 128):
            pidx = posb.at[t]
            pltpu.sync_copy(ox.at[pl.ds(base + t * 128, 128)], ssx.at[pidx])
            pltpu.sync_copy(oy.at[pl.ds(base + t * 128, 128)], ssy.at[pidx])
            pltpu.sync_copy(oz.at[pl.ds(base + t * 128, 128)], ssz.at[pidx])
            pltpu.sync_copy(idxsh.at[pl.ds(t * 128, 128)], ssidx.at[pidx])
        plsc.subcore_barrier()

        pltpu.sync_copy(ssx.at[pl.ds(0, P)], sx)
        pltpu.sync_copy(ssy.at[pl.ds(0, P)], sy)
        pltpu.sync_copy(ssz.at[pl.ds(0, P)], sz)
        pltpu.sync_copy(ssidx.at[pl.ds(0, P)], sidxb)

        DX4 = IOTA & 3
        DY4 = IOTA >> 2

        def query_body(qi, carry):
            qsplat = Z16 + qi
            qxv = plsc.load_gather(qxb, [qsplat])
            qyv = plsc.load_gather(qyb, [qsplat])
            qzv = plsc.load_gather(qzb, [qsplat])
            xlo = _cell1d(qxv - MARGIN)
            xhi = _cell1d(qxv + MARGIN)
            ylo = _cell1d(qyv - MARGIN)
            yhi = _cell1d(qyv + MARGIN)
            zlo = _cell1d(qzv - MARGIN)
            zhi = _cell1d(qzv + MARGIN)
            colx = xlo + DX4
            coly = ylo + DY4
            colvalid = (colx <= xhi) & (coly <= yhi)
            startid = colx * 100 + coly * 10 + zlo
            endid = startid + (zhi - zlo) + 1
            sts = plsc.load_gather(offs, [jnp.where(colvalid, startid, 0)])
            ens = plsc.load_gather(offs, [jnp.where(colvalid, endid, 0)])
            sts = jnp.where(colvalid, sts, 0)
            ens = jnp.where(colvalid, ens, 0)

            cnt = jnp.int32(0)
            for r in range(16):
                lmask = IOTA == r
                sr = jnp.sum(jnp.where(lmask, sts, 0))
                er = jnp.sum(jnp.where(lmask, ens, 0))

                def cand_body(t, cnt_in):
                    cand = sr + t * 16 + IOTA
                    ml = cand < er
                    csafe = jnp.where(ml, cand, 0)
                    sxv = plsc.load_gather(sx, [csafe])
                    syv = plsc.load_gather(sy, [csafe])
                    szv = plsc.load_gather(sz, [csafe])
                    siv = plsc.load_gather(sidxb, [csafe])
                    d0 = qxv - sxv
                    d1 = qyv - syv
                    d2 = qzv - szv
                    dd = (d0 * d0 + d1 * d1) + d2 * d2
                    w = (dd <= R2) & ml
                    incl = plsc.cumsum(w.astype(i32))
                    pos = jnp.where(w, cnt_in + incl - 1, P - 1)
                    plsc.store_scatter(surv, [pos], siv, mask=w)
                    return cnt_in + jnp.max(incl)

                cnt = lax.fori_loop(0, (er - sr + 15) >> 4, cand_body, cnt)

            def sel_body(ch, carry_b):
                b0, b1 = carry_b
                ci = ch * 16 + IOTA
                mm = ci < cnt
                sv = plsc.load_gather(surv, [jnp.where(mm, ci, 0)])
                cs = jnp.sort(jnp.where(mm, sv, SENT))
                lo = jnp.sort(jnp.minimum(b1, jnp.flip(cs)))
                rs = jnp.flip(lo)
                b0n = jnp.sort(jnp.minimum(b0, rs))
                b1n = jnp.sort(jnp.maximum(b0, rs))
                return b0n, b1n

            b0, b1 = lax.fori_loop(
                0, (cnt + 15) >> 4, sel_body,
                (jnp.full((16,), SENT, i32), jnp.full((16,), SENT, i32)))

            v0 = b0 < SENT
            v1 = b1 < SENT
            map0 = jnp.where(v0, b0, 0)
            map1 = jnp.where(v1, b1, 0)
            num = jnp.sum(v0.astype(i32)) + jnp.sum(v1.astype(i32))
            plsc.store_scatter(numb, [qsplat], Z16 + num, mask=IOTA == 0)
            mrow[pl.ds(0, 16)] = map0
            mrow[pl.ds(16, 16)] = map1
            gx0 = jnp.where(v0, plsc.load_gather(ox, [map0]), jnp.float32(0))
            gy0 = jnp.where(v0, plsc.load_gather(oy, [map0]), jnp.float32(0))
            gz0 = jnp.where(v0, plsc.load_gather(oz, [map0]), jnp.float32(0))
            gx1 = jnp.where(v1, plsc.load_gather(ox, [map1]), jnp.float32(0))
            gy1 = jnp.where(v1, plsc.load_gather(oy, [map1]), jnp.float32(0))
            gz1 = jnp.where(v1, plsc.load_gather(oz, [map1]), jnp.float32(0))
            p3 = IOTA * 3
            plsc.store_scatter(orow, [p3], gx0)
            plsc.store_scatter(orow, [p3 + 1], gy0)
            plsc.store_scatter(orow, [p3 + 2], gz0)
            plsc.store_scatter(orow, [p3 + 48], gx1)
            plsc.store_scatter(orow, [p3 + 49], gy1)
            plsc.store_scatter(orow, [p3 + 50], gz1)
            qg = wid * QW + qi

            @pl.when(qg < N1)
            def _():
                pltpu.sync_copy(mrow, map_o.at[qg])
                pltpu.sync_copy(orow, out_o.at[qg])

            return carry

        lax.fori_loop(0, QW, query_body, 0)
        pltpu.sync_copy(numb, num_o.at[pl.ds(wid * QW, QW)])

    return kfn(qx, qy, qz, px, py, pz)


@jax.jit
def kernel(points1, points2, lengths1, lengths2):
    p1 = points1[0]
    p2 = points2[0]
    padq = ((0, P - N1),)
    padp = ((0, P - N2),)
    qx = jnp.pad(p1[:, 0], padq, constant_values=2.0)
    qy = jnp.pad(p1[:, 1], padq, constant_values=2.0)
    qz = jnp.pad(p1[:, 2], padq, constant_values=2.0)
    px = jnp.pad(p2[:, 0], padp)
    py = jnp.pad(p2[:, 1], padp)
    pz = jnp.pad(p2[:, 2], padp)
    map_o, num_o, out_o = _sc_ball_query(qx, qy, qz, px, py, pz)
    mapping = map_o[:N1][None]
    num = num_o[:N1][None]
    outputs = out_o[:N1].reshape(N1, K, 3)[None]
    return mapping, num, outputs


# speedup vs baseline: 5.1337x; 1.0188x over previous
import functools

import jax
import jax.numpy as jnp
import numpy as np
from jax import lax
from jax.experimental import pallas as pl
from jax.experimental.pallas import tpu as pltpu
from jax.experimental.pallas import tpu_sc as plsc

K = 32
N1 = 10000
N2 = 10000
P = 10240
SHARD = 640
NCHUNK = SHARD // 16
NCELL = 1008
QW = P // 32
SENT = 1 << 30
R2 = np.float32(0.01)
MARGIN = np.float32(0.100002)


def _cell1d(v):
    return jnp.clip((v * jnp.float32(10.0)).astype(jnp.int32), 0, 9)


def _sc_ball_query(qx, qy, qz, px, py, pz):
    mesh = plsc.VectorSubcoreMesh(core_axis_name="c", subcore_axis_name="s")
    f32, i32 = jnp.float32, jnp.int32

    @functools.partial(
        pl.kernel,
        mesh=mesh,
        compiler_params=pltpu.CompilerParams(needs_layout_passes=False),
        out_type=[
            jax.ShapeDtypeStruct((P, K), i32),
            jax.ShapeDtypeStruct((P,), i32),
            jax.ShapeDtypeStruct((P, 3 * K), f32),
        ],
        scratch_types=[
            pltpu.VMEM((P,), f32),
            pltpu.VMEM((P,), f32),
            pltpu.VMEM((P,), f32),
            pltpu.VMEM((P,), f32),
            pltpu.VMEM((P,), f32),
            pltpu.VMEM((P,), f32),
            pltpu.VMEM((P,), i32),
            pltpu.VMEM((P,), i32),
            pltpu.VMEM((16, NCELL), i32),
            pltpu.VMEM((NCELL,), i32),
            pltpu.VMEM((NCELL,), i32),
            pltpu.VMEM((SHARD,), i32),
            pltpu.VMEM((SHARD,), i32),
            pltpu.VMEM((SHARD,), i32),
            pltpu.VMEM((SHARD // 128, 128), i32),
            pltpu.VMEM((SHARD,), i32),
            pltpu.VMEM((QW,), f32),
            pltpu.VMEM((QW,), f32),
            pltpu.VMEM((QW,), f32),
            pltpu.VMEM((QW,), i32),
            pltpu.VMEM((64, K), i32),
            pltpu.VMEM((64, 3 * K), f32),
            pltpu.VMEM_SHARED((P + 4096,), f32),
            pltpu.VMEM_SHARED((P + 4096,), f32),
            pltpu.VMEM_SHARED((P + 4096,), f32),
            pltpu.VMEM_SHARED((P + 4096,), i32),
            pltpu.VMEM_SHARED((24, NCELL), i32),
            pltpu.VMEM_SHARED((16384,), i32),
        ],
    )
    def kfn(qx_h, qy_h, qz_h, px_h, py_h, pz_h,
            map_o, num_o, out_o,
            ox, oy, oz, sx, sy, sz, sidxb, surv, hist16, offs, cur,
            cellb, priorb, totb, posb, idxsh,
            qxb, qyb, qzb, numb, mrows, orows,
            ssx, ssy, ssz, ssidx, shist, spad):
        c = lax.axis_index("c")
        s = lax.axis_index("s")
        wid = c * 16 + s
        IOTA = lax.iota(i32, 16)
        Z16 = jnp.zeros((16,), i32)

        def zero_body(t, carry):
            plsc.store_scatter(cur, [t * 16 + IOTA], Z16)
            return carry

        lax.fori_loop(0, NCELL // 16, zero_body, 0)

        pltpu.sync_copy(px_h, ox)
        pltpu.sync_copy(py_h, oy)
        pltpu.sync_copy(pz_h, oz)
        pltpu.sync_copy(qx_h.at[pl.ds(wid * QW, QW)], qxb)
        pltpu.sync_copy(qy_h.at[pl.ds(wid * QW, QW)], qyb)
        pltpu.sync_copy(qz_h.at[pl.ds(wid * QW, QW)], qzb)

        def hist_body(j, carry):
            loc = j * 16 + IOTA
            gv = s * SHARD + loc
            m = gv < N2
            xv = plsc.load_gather(ox, [gv])
            yv = plsc.load_gather(oy, [gv])
            zv = plsc.load_gather(oz, [gv])
            cellv = _cell1d(xv) * 100 + _cell1d(yv) * 10 + _cell1d(zv)
            key = jnp.where(m, cellv, -1 - IOTA)
            prior = Z16
            after = Z16
            for k in range(1, 16):
                fwd = jnp.take_along_axis(
                    key, jnp.clip(IOTA - k, 0, 15), axis=0,
                    mode="promise_in_bounds")
                bwd = jnp.take_along_axis(
                    key, jnp.clip(IOTA + k, 0, 15), axis=0,
                    mode="promise_in_bounds")
                prior = prior + jnp.where((fwd == key) & (IOTA >= k), 1, 0)
                after = after + jnp.where((bwd == key) & (IOTA <= 15 - k), 1, 0)
            tot = prior + after + 1
            cellsafe = jnp.where(m, cellv, 1000 + IOTA)
            hv = plsc.load_gather(cur, [cellsafe])
            plsc.store_scatter(cur, [cellsafe], hv + tot, mask=m)
            plsc.store_scatter(cellb, [loc], cellsafe)
            plsc.store_scatter(priorb, [loc], prior)
            plsc.store_scatter(totb, [loc], tot)
            plsc.store_scatter(idxsh, [loc], gv)
            return carry

        lax.fori_loop(0, NCHUNK, hist_body, 0)

        pltpu.sync_copy(cur, shist.at[s])
        plsc.subcore_barrier()
        pltpu.sync_copy(shist.at[pl.ds(0, 16)], hist16)

        def offs_body(t, run):
            bidx = t * 16 + IOTA
            hsum = Z16
            prior_s = Z16
            for s2 in range(16):
                h = plsc.load_gather(hist16, [Z16 + s2, bidx])
                hsum = hsum + h
                prior_s = prior_s + jnp.where(s2 < s, h, 0)
            incl = plsc.cumsum(hsum)
            excl = incl - hsum
            plsc.store_scatter(offs, [bidx], run + excl)
            plsc.store_scatter(cur, [bidx], run + excl + prior_s)
            return run + jnp.max(incl)

        lax.fori_loop(0, NCELL // 16, offs_body, jnp.int32(0))

        def pos_body(j, carry):
            loc = j * 16 + IOTA
            gv = s * SHARD + loc
            m = gv < N2
            cellsafe = plsc.load_gather(cellb, [loc])
            prior = plsc.load_gather(priorb, [loc])
            tot = plsc.load_gather(totb, [loc])
            c0 = plsc.load_gather(cur, [cellsafe])
            pos = jnp.where(m, c0 + prior, P - 16 + IOTA)
            plsc.store_scatter(posb, [loc >> 7, loc & 127], pos)
            plsc.store_scatter(cur, [cellsafe], c0 + tot, mask=m)
            return carry

        lax.fori_loop(0, NCHUNK, pos_body, 0)

        base = s * SHARD
        for t in range(SHARD // 128):
            pidx = posb.at[t]
            pltpu.sync_copy(ox.at[pl.ds(base + t * 128, 128)], ssx.at[pidx])
            pltpu.sync_copy(oy.at[pl.ds(base + t * 128, 128)], ssy.at[pidx])
            pltpu.sync_copy(oz.at[pl.ds(base + t * 128, 128)], ssz.at[pidx])
            pltpu.sync_copy(idxsh.at[pl.ds(t * 128, 128)], ssidx.at[pidx])
        plsc.subcore_barrier()

        pltpu.sync_copy(ssx.at[pl.ds(0, P)], sx)
        pltpu.sync_copy(ssy.at[pl.ds(0, P)], sy)
        pltpu.sync_copy(ssz.at[pl.ds(0, P)], sz)
        pltpu.sync_copy(ssidx.at[pl.ds(0, P)], sidxb)

        DX4 = IOTA & 3
        DY4 = IOTA >> 2

        def query_body(qi, carry):
            bq = carry
            qsplat = Z16 + qi
            qxv = plsc.load_gather(qxb, [qsplat])
            qyv = plsc.load_gather(qyb, [qsplat])
            qzv = plsc.load_gather(qzb, [qsplat])
            xlo = _cell1d(qxv - MARGIN)
            xhi = _cell1d(qxv + MARGIN)
            ylo = _cell1d(qyv - MARGIN)
            yhi = _cell1d(qyv + MARGIN)
            zlo = _cell1d(qzv - MARGIN)
            zhi = _cell1d(qzv + MARGIN)
            colx = xlo + DX4
            coly = ylo + DY4
            colvalid = (colx <= xhi) & (coly <= yhi)
            startid = colx * 100 + coly * 10 + zlo
            endid = startid + (zhi - zlo) + 1
            sts = plsc.load_gather(offs, [jnp.where(colvalid, startid, 0)])
            ens = plsc.load_gather(offs, [jnp.where(colvalid, endid, 0)])
            sts = jnp.where(colvalid, sts, 0)
            ens = jnp.where(colvalid, ens, 0)

            cnt = jnp.int32(0)
            for r in range(16):
                sr = sts[r]
                er = ens[r]

                def cand_body(t, cnt_in):
                    cand = sr + t * 16 + IOTA
                    ml = cand < er
                    csafe = jnp.where(ml, cand, 0)
                    sxv = plsc.load_gather(sx, [csafe])
                    syv = plsc.load_gather(sy, [csafe])
                    szv = plsc.load_gather(sz, [csafe])
                    siv = plsc.load_gather(sidxb, [csafe])
                    d0 = qxv - sxv
                    d1 = qyv - syv
                    d2 = qzv - szv
                    dd = (d0 * d0 + d1 * d1) + d2 * d2
                    w = (dd <= R2) & ml
                    incl = plsc.cumsum(w.astype(i32))
                    pos = jnp.where(w, cnt_in + incl - 1, P - 1)
                    plsc.store_scatter(surv, [pos], siv, mask=w)
                    return cnt_in + jnp.max(incl)

                cnt = lax.fori_loop(0, (er - sr + 15) >> 4, cand_body, cnt)

            def sel_body(ch, carry_b):
                b0, b1 = carry_b
                ci = ch * 16 + IOTA
                mm = ci < cnt
                sv = plsc.load_gather(surv, [jnp.where(mm, ci, 0)])
                cs = jnp.sort(jnp.where(mm, sv, SENT))
                lo = jnp.sort(jnp.minimum(b1, jnp.flip(cs)))
                rs = jnp.flip(lo)
                b0n = jnp.sort(jnp.minimum(b0, rs))
                b1n = jnp.sort(jnp.maximum(b0, rs))
                return b0n, b1n

            b0, b1 = lax.fori_loop(
                0, (cnt + 15) >> 4, sel_body,
                (jnp.full((16,), SENT, i32), jnp.full((16,), SENT, i32)))

            v0 = b0 < SENT
            v1 = b1 < SENT
            map0 = jnp.where(v0, b0, 0)
            map1 = jnp.where(v1, b1, 0)
            num = jnp.sum(v0.astype(i32)) + jnp.sum(v1.astype(i32))
            plsc.store_scatter(numb, [qsplat], Z16 + num, mask=IOTA == 0)
            bsplat = Z16 + bq
            plsc.store_scatter(mrows, [bsplat, IOTA], map0)
            plsc.store_scatter(mrows, [bsplat, IOTA + 16], map1)
            gx0 = jnp.where(v0, plsc.load_gather(ox, [map0]), jnp.float32(0))
            gy0 = jnp.where(v0, plsc.load_gather(oy, [map0]), jnp.float32(0))
            gz0 = jnp.where(v0, plsc.load_gather(oz, [map0]), jnp.float32(0))
            gx1 = jnp.where(v1, plsc.load_gather(ox, [map1]), jnp.float32(0))
            gy1 = jnp.where(v1, plsc.load_gather(oy, [map1]), jnp.float32(0))
            gz1 = jnp.where(v1, plsc.load_gather(oz, [map1]), jnp.float32(0))
            p3 = IOTA * 3
            plsc.store_scatter(orows, [bsplat, p3], gx0)
            plsc.store_scatter(orows, [bsplat, p3 + 1], gy0)
            plsc.store_scatter(orows, [bsplat, p3 + 2], gz0)
            plsc.store_scatter(orows, [bsplat, p3 + 48], gx1)
            plsc.store_scatter(orows, [bsplat, p3 + 49], gy1)
            plsc.store_scatter(orows, [bsplat, p3 + 50], gz1)
            return bq + 1

        def batch_body(b, carry):
            lax.fori_loop(b * 64, b * 64 + 64, query_body, jnp.int32(0))
            rowbase = wid * QW + b * 64
            pltpu.sync_copy(mrows, map_o.at[pl.ds(rowbase, 64)])
            pltpu.sync_copy(orows, out_o.at[pl.ds(rowbase, 64)])
            return carry

        lax.fori_loop(0, QW // 64, batch_body, 0)
        pltpu.sync_copy(numb, num_o.at[pl.ds(wid * QW, QW)])

    return kfn(qx, qy, qz, px, py, pz)


@jax.jit
def kernel(points1, points2, lengths1, lengths2):
    p1 = points1[0]
    p2 = points2[0]
    padq = ((0, P - N1),)
    padp = ((0, P - N2),)
    qx = jnp.pad(p1[:, 0], padq, constant_values=2.0)
    qy = jnp.pad(p1[:, 1], padq, constant_values=2.0)
    qz = jnp.pad(p1[:, 2], padq, constant_values=2.0)
    px = jnp.pad(p2[:, 0], padp)
    py = jnp.pad(p2[:, 1], padp)
    pz = jnp.pad(p2[:, 2], padp)
    map_o, num_o, out_o = _sc_ball_query(qx, qy, qz, px, py, pz)
    mapping = map_o[:N1][None]
    num = num_o[:N1][None]
    outputs = out_o[:N1].reshape(N1, K, 3)[None]
    return mapping, num, outputs


# speedup vs baseline: 5.6329x; 1.0972x over previous
import functools

import jax
import jax.numpy as jnp
import numpy as np
from jax import lax
from jax.experimental import pallas as pl
from jax.experimental.pallas import tpu as pltpu
from jax.experimental.pallas import tpu_sc as plsc

K = 32
N1 = 10000
N2 = 10000
P = 10240
SHARD = 640
NCHUNK = SHARD // 16
NCELL = 1008
QW = P // 32
SENT = 1 << 30
R2 = np.float32(0.01)
MARGIN = np.float32(0.100002)


def _cell1d(v):
    return jnp.clip((v * jnp.float32(10.0)).astype(jnp.int32), 0, 9)


def _sc_ball_query(qx, qy, qz, px, py, pz):
    mesh = plsc.VectorSubcoreMesh(core_axis_name="c", subcore_axis_name="s")
    f32, i32 = jnp.float32, jnp.int32

    @functools.partial(
        pl.kernel,
        mesh=mesh,
        compiler_params=pltpu.CompilerParams(needs_layout_passes=False),
        out_type=[
            jax.ShapeDtypeStruct((P, K), i32),
            jax.ShapeDtypeStruct((P,), i32),
            jax.ShapeDtypeStruct((P, 3 * K), f32),
        ],
        scratch_types=[
            pltpu.VMEM((P,), f32),
            pltpu.VMEM((P,), f32),
            pltpu.VMEM((P,), f32),
            pltpu.VMEM((P,), f32),
            pltpu.VMEM((P,), f32),
            pltpu.VMEM((P,), f32),
            pltpu.VMEM((P,), i32),
            pltpu.VMEM((P,), i32),
            pltpu.VMEM((16, NCELL), i32),
            pltpu.VMEM((NCELL,), i32),
            pltpu.VMEM((NCELL,), i32),
            pltpu.VMEM((SHARD,), i32),
            pltpu.VMEM((SHARD,), i32),
            pltpu.VMEM((SHARD,), i32),
            pltpu.VMEM((SHARD // 128, 128), i32),
            pltpu.VMEM((SHARD,), i32),
            pltpu.VMEM((QW,), f32),
            pltpu.VMEM((QW,), f32),
            pltpu.VMEM((QW,), f32),
            pltpu.VMEM((QW,), i32),
            pltpu.VMEM((64, K), i32),
            pltpu.VMEM((64, 3 * K), f32),
            pltpu.VMEM_SHARED((P + 4096,), f32),
            pltpu.VMEM_SHARED((P + 4096,), f32),
            pltpu.VMEM_SHARED((P + 4096,), f32),
            pltpu.VMEM_SHARED((P + 4096,), i32),
            pltpu.VMEM_SHARED((24, NCELL), i32),
            pltpu.VMEM_SHARED((16384,), i32),
        ],
    )
    def kfn(qx_h, qy_h, qz_h, px_h, py_h, pz_h,
            map_o, num_o, out_o,
            ox, oy, oz, sx, sy, sz, sidxb, surv, hist16, offs, cur,
            cellb, priorb, totb, posb, idxsh,
            qxb, qyb, qzb, numb, mrows, orows,
            ssx, ssy, ssz, ssidx, shist, spad):
        c = lax.axis_index("c")
        s = lax.axis_index("s")
        wid = c * 16 + s
        IOTA = lax.iota(i32, 16)
        Z16 = jnp.zeros((16,), i32)

        def zero_body(t, carry):
            plsc.store_scatter(cur, [t * 16 + IOTA], Z16)
            return carry

        lax.fori_loop(0, NCELL // 16, zero_body, 0)

        pltpu.sync_copy(px_h, ox)
        pltpu.sync_copy(py_h, oy)
        pltpu.sync_copy(pz_h, oz)
        pltpu.sync_copy(qx_h.at[pl.ds(wid * QW, QW)], qxb)
        pltpu.sync_copy(qy_h.at[pl.ds(wid * QW, QW)], qyb)
        pltpu.sync_copy(qz_h.at[pl.ds(wid * QW, QW)], qzb)

        def hist_body(j, carry):
            loc = j * 16 + IOTA
            gv = s * SHARD + loc
            m = gv < N2
            xv = plsc.load_gather(ox, [gv])
            yv = plsc.load_gather(oy, [gv])
            zv = plsc.load_gather(oz, [gv])
            cellv = _cell1d(xv) * 100 + _cell1d(yv) * 10 + _cell1d(zv)
            key = jnp.where(m, cellv, -1 - IOTA)
            prior = Z16
            after = Z16
            for k in range(1, 16):
                fwd = jnp.take_along_axis(
                    key, jnp.clip(IOTA - k, 0, 15), axis=0,
                    mode="promise_in_bounds")
                bwd = jnp.take_along_axis(
                    key, jnp.clip(IOTA + k, 0, 15), axis=0,
                    mode="promise_in_bounds")
                prior = prior + jnp.where((fwd == key) & (IOTA >= k), 1, 0)
                after = after + jnp.where((bwd == key) & (IOTA <= 15 - k), 1, 0)
            tot = prior + after + 1
            cellsafe = jnp.where(m, cellv, 1000 + IOTA)
            hv = plsc.load_gather(cur, [cellsafe])
            plsc.store_scatter(cur, [cellsafe], hv + tot, mask=m)
            plsc.store_scatter(cellb, [loc], cellsafe)
            plsc.store_scatter(priorb, [loc], prior)
            plsc.store_scatter(totb, [loc], tot)
            plsc.store_scatter(idxsh, [loc], gv)
            return carry

        lax.fori_loop(0, NCHUNK, hist_body, 0)

        pltpu.sync_copy(cur, shist.at[s])
        plsc.subcore_barrier()
        pltpu.sync_copy(shist.at[pl.ds(0, 16)], hist16)

        def offs_body(t, run):
            bidx = t * 16 + IOTA
            hsum = Z16
            prior_s = Z16
            for s2 in range(16):
                h = plsc.load_gather(hist16, [Z16 + s2, bidx])
                hsum = hsum + h
                prior_s = prior_s + jnp.where(s2 < s, h, 0)
            incl = plsc.cumsum(hsum)
            excl = incl - hsum
            plsc.store_scatter(offs, [bidx], run + excl)
            plsc.store_scatter(cur, [bidx], run + excl + prior_s)
            return run + jnp.max(incl)

        lax.fori_loop(0, NCELL // 16, offs_body, jnp.int32(0))

        def pos_body(j, carry):
            loc = j * 16 + IOTA
            gv = s * SHARD + loc
            m = gv < N2
            cellsafe = plsc.load_gather(cellb, [loc])
            prior = plsc.load_gather(priorb, [loc])
            tot = plsc.load_gather(totb, [loc])
            c0 = plsc.load_gather(cur, [cellsafe])
            pos = jnp.where(m, c0 + prior, P - 16 + IOTA)
            plsc.store_scatter(posb, [loc >> 7, loc & 127], pos)
            plsc.store_scatter(cur, [cellsafe], c0 + tot, mask=m)
            return carry

        lax.fori_loop(0, NCHUNK, pos_body, 0)

        base = s * SHARD
        for t in range(SHARD // 128):
            pidx = posb.at[t]
            pltpu.sync_copy(ox.at[pl.ds(base + t * 128, 128)], ssx.at[pidx])
            pltpu.sync_copy(oy.at[pl.ds(base + t * 128, 128)], ssy.at[pidx])
            pltpu.sync_copy(oz.at[pl.ds(base + t * 128, 128)], ssz.at[pidx])
            pltpu.sync_copy(idxsh.at[pl.ds(t * 128, 128)], ssidx.at[pidx])
        plsc.subcore_barrier()

        pltpu.sync_copy(ssx.at[pl.ds(0, P)], sx)
        pltpu.sync_copy(ssy.at[pl.ds(0, P)], sy)
        pltpu.sync_copy(ssz.at[pl.ds(0, P)], sz)
        pltpu.sync_copy(ssidx.at[pl.ds(0, P)], sidxb)

        DX4 = IOTA & 3
        DY4 = IOTA >> 2

        def query_body(qi, carry):
            bq = carry
            qsplat = Z16 + qi
            qxv = plsc.load_gather(qxb, [qsplat])
            qyv = plsc.load_gather(qyb, [qsplat])
            qzv = plsc.load_gather(qzb, [qsplat])
            xlo = _cell1d(qxv - MARGIN)
            xhi = _cell1d(qxv + MARGIN)
            ylo = _cell1d(qyv - MARGIN)
            yhi = _cell1d(qyv + MARGIN)
            zlo = _cell1d(qzv - MARGIN)
            zhi = _cell1d(qzv + MARGIN)
            colx = xlo + DX4
            coly = ylo + DY4
            colvalid = (colx <= xhi) & (coly <= yhi)
            startid = colx * 100 + coly * 10 + zlo
            endid = startid + (zhi - zlo) + 1
            sts = plsc.load_gather(offs, [jnp.where(colvalid, startid, 0)])
            ens = plsc.load_gather(offs, [jnp.where(colvalid, endid, 0)])
            sts = jnp.where(colvalid, sts, 0)
            ens = jnp.where(colvalid, ens, 0)

            cnt = jnp.int32(0)
            for r in range(16):
                sr = sts[r]
                er = ens[r]

                def cand_body(t, cnt_in):
                    cand = sr + t * 16 + IOTA
                    ml = cand < er
                    csafe = jnp.where(ml, cand, 0)
                    sxv = plsc.load_gather(sx, [csafe])
                    syv = plsc.load_gather(sy, [csafe])
                    szv = plsc.load_gather(sz, [csafe])
                    siv = plsc.load_gather(sidxb, [csafe])
                    d0 = qxv - sxv
                    d1 = qyv - syv
                    d2 = qzv - szv
                    dd = (d0 * d0 + d1 * d1) + d2 * d2
                    w = (dd <= R2) & ml
                    tot = plsc.all_reduce_population_count(w)
                    incl = plsc.cumsum(w.astype(i32))
                    pos = jnp.where(w, cnt_in + incl - 1, P - 1)
                    plsc.store_scatter(surv, [pos], siv, mask=w)
                    return cnt_in + tot[0]

                cnt = lax.fori_loop(0, (er - sr + 15) >> 4, cand_body, cnt)

            def sel_body(ch, carry_b):
                b0, b1 = carry_b
                ci = ch * 16 + IOTA
                mm = ci < cnt
                sv = plsc.load_gather(surv, [jnp.where(mm, ci, 0)])
                cs = jnp.sort(jnp.where(mm, sv, SENT))
                lo = jnp.sort(jnp.minimum(b1, jnp.flip(cs)))
                rs = jnp.flip(lo)
                b0n = jnp.sort(jnp.minimum(b0, rs))
                b1n = jnp.sort(jnp.maximum(b0, rs))
                return b0n, b1n

            b0, b1 = lax.fori_loop(
                0, (cnt + 15) >> 4, sel_body,
                (jnp.full((16,), SENT, i32), jnp.full((16,), SENT, i32)))

            v0 = b0 < SENT
            v1 = b1 < SENT
            map0 = jnp.where(v0, b0, 0)
            map1 = jnp.where(v1, b1, 0)
            num = jnp.sum(v0.astype(i32)) + jnp.sum(v1.astype(i32))
            plsc.store_scatter(numb, [qsplat], Z16 + num, mask=IOTA == 0)
            bsplat = Z16 + bq
            plsc.store_scatter(mrows, [bsplat, IOTA], map0)
            plsc.store_scatter(mrows, [bsplat, IOTA + 16], map1)
            gx0 = jnp.where(v0, plsc.load_gather(ox, [map0]), jnp.float32(0))
            gy0 = jnp.where(v0, plsc.load_gather(oy, [map0]), jnp.float32(0))
            gz0 = jnp.where(v0, plsc.load_gather(oz, [map0]), jnp.float32(0))
            gx1 = jnp.where(v1, plsc.load_gather(ox, [map1]), jnp.float32(0))
            gy1 = jnp.where(v1, plsc.load_gather(oy, [map1]), jnp.float32(0))
            gz1 = jnp.where(v1, plsc.load_gather(oz, [map1]), jnp.float32(0))
            p3 = IOTA * 3
            plsc.store_scatter(orows, [bsplat, p3], gx0)
            plsc.store_scatter(orows, [bsplat, p3 + 1], gy0)
            plsc.store_scatter(orows, [bsplat, p3 + 2], gz0)
            plsc.store_scatter(orows, [bsplat, p3 + 48], gx1)
            plsc.store_scatter(orows, [bsplat, p3 + 49], gy1)
            plsc.store_scatter(orows, [bsplat, p3 + 50], gz1)
            return bq + 1

        def batch_body(b, carry):
            lax.fori_loop(b * 64, b * 64 + 64, query_body, jnp.int32(0))
            rowbase = wid * QW + b * 64
            pltpu.sync_copy(mrows, map_o.at[pl.ds(rowbase, 64)])
            pltpu.sync_copy(orows, out_o.at[pl.ds(rowbase, 64)])
            return carry

        lax.fori_loop(0, QW // 64, batch_body, 0)
        pltpu.sync_copy(numb, num_o.at[pl.ds(wid * QW, QW)])

    return kfn(qx, qy, qz, px, py, pz)


@jax.jit
def kernel(points1, points2, lengths1, lengths2):
    p1 = points1[0]
    p2 = points2[0]
    padq = ((0, P - N1),)
    padp = ((0, P - N2),)
    qx = jnp.pad(p1[:, 0], padq, constant_values=2.0)
    qy = jnp.pad(p1[:, 1], padq, constant_values=2.0)
    qz = jnp.pad(p1[:, 2], padq, constant_values=2.0)
    px = jnp.pad(p2[:, 0], padp)
    py = jnp.pad(p2[:, 1], padp)
    pz = jnp.pad(p2[:, 2], padp)
    map_o, num_o, out_o = _sc_ball_query(qx, qy, qz, px, py, pz)
    mapping = map_o[:N1][None]
    num = num_o[:N1][None]
    outputs = out_o[:N1].reshape(N1, K, 3)[None]
    return mapping, num, outputs


# speedup vs baseline: 10.9283x; 1.9401x over previous
import functools

import jax
import jax.numpy as jnp
import numpy as np
from jax import lax
from jax.experimental import pallas as pl
from jax.experimental.pallas import tpu as pltpu
from jax.experimental.pallas import tpu_sc as plsc

K = 32
N1 = 10000
N2 = 10000
P = 10240
SHARD = 640
NCHUNK = SHARD // 16
NCELL = 1008
QW = P // 32
SENT = 1 << 30
R2 = np.float32(0.01)
MARGIN = np.float32(0.100002)


def _cell1d(v):
    return jnp.clip((v * jnp.float32(10.0)).astype(jnp.int32), 0, 9)


def _sc_ball_query(qx, qy, qz, px, py, pz):
    mesh = plsc.VectorSubcoreMesh(core_axis_name="c", subcore_axis_name="s")
    f32, i32 = jnp.float32, jnp.int32

    @functools.partial(
        pl.kernel,
        mesh=mesh,
        compiler_params=pltpu.CompilerParams(needs_layout_passes=False),
        out_type=[
            jax.ShapeDtypeStruct((P, K), i32),
            jax.ShapeDtypeStruct((P,), i32),
            jax.ShapeDtypeStruct((P, 3 * K), f32),
        ],
        scratch_types=[
            pltpu.VMEM((SHARD,), f32),
            pltpu.VMEM((SHARD,), f32),
            pltpu.VMEM((SHARD,), f32),
            pltpu.VMEM((P,), i32),
            pltpu.VMEM((P,), f32),
            pltpu.VMEM((P,), f32),
            pltpu.VMEM((P,), f32),
            pltpu.VMEM((P,), i32),
            pltpu.VMEM((P,), i32),
            pltpu.VMEM((P,), i32),
            pltpu.VMEM((16, NCELL), i32),
            pltpu.VMEM((NCELL,), i32),
            pltpu.VMEM((NCELL,), i32),
            pltpu.VMEM((SHARD,), i32),
            pltpu.VMEM((SHARD,), i32),
            pltpu.VMEM((SHARD,), i32),
            pltpu.VMEM((SHARD // 128, 128), i32),
            pltpu.VMEM((SHARD,), i32),
            pltpu.VMEM((QW,), f32),
            pltpu.VMEM((QW,), f32),
            pltpu.VMEM((QW,), f32),
            pltpu.VMEM((QW,), i32),
            pltpu.VMEM((64, K), i32),
            pltpu.VMEM((64, 3 * K), f32),
            pltpu.VMEM_SHARED((P + 4096,), f32),
            pltpu.VMEM_SHARED((P + 4096,), f32),
            pltpu.VMEM_SHARED((P + 4096,), f32),
            pltpu.VMEM_SHARED((P + 4096,), i32),
            pltpu.VMEM_SHARED((24, NCELL), i32),
            pltpu.VMEM_SHARED((16384,), i32),
        ],
    )
    def kfn(qx_h, qy_h, qz_h, px_h, py_h, pz_h,
            map_o, num_o, out_o,
            tx, ty, tz, invp, sx, sy, sz, sidxb, surv, survb, hist16, offs, cur,
            cellb, priorb, totb, posb, idxsh,
            qxb, qyb, qzb, numb, mrows, orows,
            ssx, ssy, ssz, ssidx, shist, spad):
        c = lax.axis_index("c")
        s = lax.axis_index("s")
        wid = c * 16 + s
        IOTA = lax.iota(i32, 16)
        Z16 = jnp.zeros((16,), i32)

        def zero_body(t, carry):
            plsc.store_scatter(cur, [t * 16 + IOTA], Z16)
            return carry

        lax.fori_loop(0, NCELL // 16, zero_body, 0)

        pltpu.sync_copy(px_h.at[pl.ds(s * SHARD, SHARD)], tx)
        pltpu.sync_copy(py_h.at[pl.ds(s * SHARD, SHARD)], ty)
        pltpu.sync_copy(pz_h.at[pl.ds(s * SHARD, SHARD)], tz)
        pltpu.sync_copy(qx_h.at[pl.ds(wid * QW, QW)], qxb)
        pltpu.sync_copy(qy_h.at[pl.ds(wid * QW, QW)], qyb)
        pltpu.sync_copy(qz_h.at[pl.ds(wid * QW, QW)], qzb)

        def hist_body(j, carry):
            loc = j * 16 + IOTA
            gv = s * SHARD + loc
            m = gv < N2
            xv = plsc.load_gather(tx, [loc])
            yv = plsc.load_gather(ty, [loc])
            zv = plsc.load_gather(tz, [loc])
            cellv = _cell1d(xv) * 100 + _cell1d(yv) * 10 + _cell1d(zv)
            key = jnp.where(m, cellv, -1 - IOTA)
            prior = Z16
            after = Z16
            for k in range(1, 16):
                fwd = jnp.take_along_axis(
                    key, jnp.clip(IOTA - k, 0, 15), axis=0,
                    mode="promise_in_bounds")
                bwd = jnp.take_along_axis(
                    key, jnp.clip(IOTA + k, 0, 15), axis=0,
                    mode="promise_in_bounds")
                prior = prior + jnp.where((fwd == key) & (IOTA >= k), 1, 0)
                after = after + jnp.where((bwd == key) & (IOTA <= 15 - k), 1, 0)
            tot = prior + after + 1
            cellsafe = jnp.where(m, cellv, 1000 + IOTA)
            hv = plsc.load_gather(cur, [cellsafe])
            plsc.store_scatter(cur, [cellsafe], hv + tot, mask=m)
            plsc.store_scatter(cellb, [loc], cellsafe)
            plsc.store_scatter(priorb, [loc], prior)
            plsc.store_scatter(totb, [loc], tot)
            plsc.store_scatter(idxsh, [loc], gv)
            return carry

        lax.fori_loop(0, NCHUNK, hist_body, 0)

        pltpu.sync_copy(cur, shist.at[s])
        plsc.subcore_barrier()
        pltpu.sync_copy(shist.at[pl.ds(0, 16)], hist16)

        def offs_body(t, run):
            bidx = t * 16 + IOTA
            hsum = Z16
            prior_s = Z16
            for s2 in range(16):
                h = plsc.load_gather(hist16, [Z16 + s2, bidx])
                hsum = hsum + h
                prior_s = prior_s + jnp.where(s2 < s, h, 0)
            incl = plsc.cumsum(hsum)
            excl = incl - hsum
            plsc.store_scatter(offs, [bidx], run + excl)
            plsc.store_scatter(cur, [bidx], run + excl + prior_s)
            return run + jnp.max(incl)

        lax.fori_loop(0, NCELL // 16, offs_body, jnp.int32(0))

        def pos_body(j, carry):
            loc = j * 16 + IOTA
            gv = s * SHARD + loc
            m = gv < N2
            cellsafe = plsc.load_gather(cellb, [loc])
            prior = plsc.load_gather(priorb, [loc])
            tot = plsc.load_gather(totb, [loc])
            c0 = plsc.load_gather(cur, [cellsafe])
            pos = jnp.where(m, c0 + prior, P - 16 + IOTA)
            plsc.store_scatter(posb, [loc >> 7, loc & 127], pos)
            plsc.store_scatter(cur, [cellsafe], c0 + tot, mask=m)
            return carry

        lax.fori_loop(0, NCHUNK, pos_body, 0)

        for t in range(SHARD // 128):
            pidx = posb.at[t]
            pltpu.sync_copy(tx.at[pl.ds(t * 128, 128)], ssx.at[pidx])
            pltpu.sync_copy(ty.at[pl.ds(t * 128, 128)], ssy.at[pidx])
            pltpu.sync_copy(tz.at[pl.ds(t * 128, 128)], ssz.at[pidx])
            pltpu.sync_copy(idxsh.at[pl.ds(t * 128, 128)], ssidx.at[pidx])
        plsc.subcore_barrier()

        pltpu.sync_copy(ssx.at[pl.ds(0, P)], sx)
        pltpu.sync_copy(ssy.at[pl.ds(0, P)], sy)
        pltpu.sync_copy(ssz.at[pl.ds(0, P)], sz)
        pltpu.sync_copy(ssidx.at[pl.ds(0, P)], sidxb)

        def invp_body(t, carry):
            ci = t * 16 + IOTA
            mm = ci < N2
            v = jnp.where(mm, plsc.load_gather(sidxb, [jnp.where(mm, ci, 0)]), 0)
            plsc.store_scatter(invp, [v], ci, mask=mm)
            return carry

        lax.fori_loop(0, N2 // 16, invp_body, 0)

        DX4 = IOTA & 3
        DY4 = IOTA >> 2

        def query_body(qi, carry):
            bq = carry
            qsplat = Z16 + qi
            qxv = plsc.load_gather(qxb, [qsplat])
            qyv = plsc.load_gather(qyb, [qsplat])
            qzv = plsc.load_gather(qzb, [qsplat])
            xlo = _cell1d(qxv - MARGIN)
            xhi = _cell1d(qxv + MARGIN)
            ylo = _cell1d(qyv - MARGIN)
            yhi = _cell1d(qyv + MARGIN)
            zlo = _cell1d(qzv - MARGIN)
            zhi = _cell1d(qzv + MARGIN)
            colx = xlo + DX4
            coly = ylo + DY4
            colvalid = (colx <= xhi) & (coly <= yhi)
            startid = colx * 100 + coly * 10 + zlo
            endid = startid + (zhi - zlo) + 1
            sts = plsc.load_gather(offs, [jnp.where(colvalid, startid, 0)])
            ens = plsc.load_gather(offs, [jnp.where(colvalid, endid, 0)])
            sts = jnp.where(colvalid, sts, 0)
            ens = jnp.where(colvalid, ens, 0)

            def scan_chunk(t, cnt_in, sr, er, sref):
                cand = sr + t * 16 + IOTA
                ml = cand < er
                csafe = jnp.where(ml, cand, 0)
                sxv = plsc.load_gather(sx, [csafe])
                syv = plsc.load_gather(sy, [csafe])
                szv = plsc.load_gather(sz, [csafe])
                siv = plsc.load_gather(sidxb, [csafe])
                d0 = qxv - sxv
                d1 = qyv - syv
                d2 = qzv - szv
                dd = (d0 * d0 + d1 * d1) + d2 * d2
                w = (dd <= R2) & ml
                tot = plsc.all_reduce_population_count(w)
                incl = plsc.cumsum(w.astype(i32))
                pos = jnp.where(w, cnt_in + incl - 1, P - 1)
                plsc.store_scatter(sref, [pos], siv, mask=w)
                return cnt_in + tot[0]

            cnta = jnp.int32(0)
            cntb = jnp.int32(0)
            for r in range(8):
                sra = sts[r]
                era = ens[r]
                srb = sts[r + 8]
                erb = ens[r + 8]

                def cand_body(t, cc):
                    ca, cb = cc
                    ca = scan_chunk(t, ca, sra, era, surv)
                    cb = scan_chunk(t, cb, srb, erb, survb)
                    return ca, cb

                nit = jnp.maximum(era - sra, erb - srb)
                cnta, cntb = lax.fori_loop(0, (nit + 15) >> 4, cand_body,
                                           (cnta, cntb))

            def make_sel(sref, cnt):
                def sel_body(ch, carry_b):
                    b0, b1 = carry_b
                    ci = ch * 16 + IOTA
                    mm = ci < cnt
                    sv = plsc.load_gather(sref, [jnp.where(mm, ci, 0)])
                    cs = jnp.sort(jnp.where(mm, sv, SENT))
                    lo = jnp.sort(jnp.minimum(b1, jnp.flip(cs)))
                    rs = jnp.flip(lo)
                    b0n = jnp.sort(jnp.minimum(b0, rs))
                    b1n = jnp.sort(jnp.maximum(b0, rs))
                    return b0n, b1n
                return sel_body

            b0, b1 = lax.fori_loop(
                0, (cnta + 15) >> 4, make_sel(surv, cnta),
                (jnp.full((16,), SENT, i32), jnp.full((16,), SENT, i32)))
            b0, b1 = lax.fori_loop(
                0, (cntb + 15) >> 4, make_sel(survb, cntb), (b0, b1))

            v0 = b0 < SENT
            v1 = b1 < SENT
            map0 = jnp.where(v0, b0, 0)
            map1 = jnp.where(v1, b1, 0)
            num = jnp.sum(v0.astype(i32)) + jnp.sum(v1.astype(i32))
            plsc.store_scatter(numb, [qsplat], Z16 + num, mask=IOTA == 0)
            bsplat = Z16 + bq
            plsc.store_scatter(mrows, [bsplat, IOTA], map0)
            plsc.store_scatter(mrows, [bsplat, IOTA + 16], map1)
            sp0 = plsc.load_gather(invp, [map0])
            sp1 = plsc.load_gather(invp, [map1])
            gx0 = jnp.where(v0, plsc.load_gather(sx, [sp0]), jnp.float32(0))
            gy0 = jnp.where(v0, plsc.load_gather(sy, [sp0]), jnp.float32(0))
            gz0 = jnp.where(v0, plsc.load_gather(sz, [sp0]), jnp.float32(0))
            gx1 = jnp.where(v1, plsc.load_gather(sx, [sp1]), jnp.float32(0))
            gy1 = jnp.where(v1, plsc.load_gather(sy, [sp1]), jnp.float32(0))
            gz1 = jnp.where(v1, plsc.load_gather(sz, [sp1]), jnp.float32(0))
            p3 = IOTA * 3
            plsc.store_scatter(orows, [bsplat, p3], gx0)
            plsc.store_scatter(orows, [bsplat, p3 + 1], gy0)
            plsc.store_scatter(orows, [bsplat, p3 + 2], gz0)
            plsc.store_scatter(orows, [bsplat, p3 + 48], gx1)
            plsc.store_scatter(orows, [bsplat, p3 + 49], gy1)
            plsc.store_scatter(orows, [bsplat, p3 + 50], gz1)
            return bq + 1

        def batch_body(b, carry):
            lax.fori_loop(b * 64, b * 64 + 64, query_body, jnp.int32(0))
            rowbase = wid * QW + b * 64
            pltpu.sync_copy(mrows, map_o.at[pl.ds(rowbase, 64)])
            pltpu.sync_copy(orows, out_o.at[pl.ds(rowbase, 64)])
            return carry

        lax.fori_loop(0, QW // 64, batch_body, 0)
        pltpu.sync_copy(numb, num_o.at[pl.ds(wid * QW, QW)])

    return kfn(qx, qy, qz, px, py, pz)


@jax.jit
def kernel(points1, points2, lengths1, lengths2):
    p1 = points1[0]
    p2 = points2[0]
    padq = ((0, P - N1),)
    padp = ((0, P - N2),)
    qx = jnp.pad(p1[:, 0], padq, constant_values=2.0)
    qy = jnp.pad(p1[:, 1], padq, constant_values=2.0)
    qz = jnp.pad(p1[:, 2], padq, constant_values=2.0)
    px = jnp.pad(p2[:, 0], padp)
    py = jnp.pad(p2[:, 1], padp)
    pz = jnp.pad(p2[:, 2], padp)
    map_o, num_o, out_o = _sc_ball_query(qx, qy, qz, px, py, pz)
    mapping = map_o[:N1][None]
    num = num_o[:N1][None]
    outputs = out_o[:N1].reshape(N1, K, 3)[None]
    return mapping, num, outputs
